# baseline (speedup 1.0000x reference)
import functools

import jax
import jax.numpy as jnp
from jax import lax
from jax.experimental import pallas as pl
from jax.experimental.pallas import tpu as pltpu
from jax.experimental.pallas import tpu_sc as plsc

_N = 100000
_E = 6400000
_NC, _NS = 2, 16
_NW = _NC * _NS
_EPW = _E // _NW
_CHUNK = 4000
_NCHUNK = _EPW // _CHUNK
_VPC = _CHUNK // 16

_SCHED = jnp.array([0.1], dtype=jnp.float32)

_MESH = plsc.VectorSubcoreMesh(core_axis_name="c", subcore_axis_name="s")


@functools.partial(
    pl.kernel,
    out_type=jax.ShapeDtypeStruct((_NW, 16), jnp.float32),
    mesh=_MESH,
    compiler_params=pltpu.CompilerParams(needs_layout_passes=False),
    scratch_types=[
        pltpu.VMEM((_N,), jnp.int32),
        pltpu.VMEM((_CHUNK,), jnp.int32),
        pltpu.VMEM((_CHUNK,), jnp.int32),
        pltpu.VMEM((_CHUNK,), jnp.int32),
        pltpu.VMEM((_CHUNK,), jnp.int32),
        pltpu.VMEM((_CHUNK,), jnp.float32),
        pltpu.VMEM((_CHUNK,), jnp.float32),
        pltpu.VMEM((16,), jnp.float32),
        pltpu.VMEM((16,), jnp.float32),
        pltpu.SemaphoreType.DMA,
        pltpu.SemaphoreType.DMA,
    ],
)
def _stress_partials(packed_hbm, i_hbm, j_hbm, dis_hbm, lr_hbm, out_hbm,
                     table_v, i0_v, i1_v, j0_v, j1_v, d0_v, d1_v,
                     lr_v, acc_v, sem0, sem1):
    cid = lax.axis_index("c")
    sid = lax.axis_index("s")
    wid = sid * _NC + cid
    base = wid * _EPW
    sems = [sem0, sem1]
    bufs = [(i0_v, j0_v, d0_v), (i1_v, j1_v, d1_v)]

    def _chunk_srcs(c):
        off = pl.multiple_of(base + c * _CHUNK, 8)
        sl = pl.ds(off, _CHUNK)
        return (i_hbm.at[sl], j_hbm.at[sl], dis_hbm.at[sl])

    def _start(c, b):
        for src, dst in zip(_chunk_srcs(c), bufs[b]):
            pltpu.async_copy(src, dst, sems[b])

    def _wait(c, b):
        for src, dst in zip(_chunk_srcs(c), bufs[b]):
            pltpu.make_async_copy(src, dst, sems[b]).wait()

    _start(0, 0)
    _start(1, 1)
    pltpu.sync_copy(packed_hbm, table_v)
    pltpu.sync_copy(lr_hbm, lr_v)
    lr = lr_v[...]
    acc_v[...] = jnp.zeros((16,), jnp.float32)

    half = jnp.float32(1.5)
    hmag = jnp.int32(0x5F3759DF)
    rmag = jnp.int32(0x7EF311C3)
    himask = jnp.int32(-65536)

    @pl.loop(0, _NCHUNK, step=2)
    def chunk_loop(c0):
        for b in range(2):
            c = c0 + b
            ib_v, jb_v, db_v = bufs[b]
            _wait(c, b)

            @plsc.parallel_loop(0, _VPC, unroll=4)
            def vec_loop(v):
                sl = pl.ds(v * 16, 16)
                iv = ib_v[sl]
                jv = jb_v[sl]
                dv = db_v[sl]
                lanes = lax.iota(jnp.int32, 16)
                iv = (iv & jnp.int32(~0xF)) | lanes
                jv = (jv & jnp.int32(~0xF)) | lanes
                wi = plsc.load_gather(table_v, [iv])
                wj = plsc.load_gather(table_v, [jv])
                xi = plsc.bitcast(wi & himask, jnp.float32)
                yi = plsc.bitcast(wi << 16, jnp.float32)
                xj = plsc.bitcast(wj & himask, jnp.float32)
                yj = plsc.bitcast(wj << 16, jnp.float32)
                dx = xi - xj
                dy = yi - yj
                s = dx * dx + jnp.float32(1e-18) + dy * dy
                r = plsc.bitcast(hmag - (plsc.bitcast(s, jnp.int32) >> 1),
                                 jnp.float32)
                r = r * (half - jnp.float32(0.5) * s * r * r)
                r = r * (half - jnp.float32(0.5) * s * r * r)
                mag = s * r
                m = jnp.maximum(dv, lr)
                q = plsc.bitcast(rmag - plsc.bitcast(m, jnp.int32), jnp.float32)
                q = q * (jnp.float32(2.0) - m * q)
                d = mag - dv
                plsc.addupdate(acc_v.at[:], (jnp.float32(0.25) * q) * (d * d))

            @pl.when(c + 2 < _NCHUNK)
            def _prefetch():
                _start(c + 2, b)

    pltpu.sync_copy(acc_v, out_hbm.at[wid])


def kernel(pos, i, j, vis_p_i, vis_p_j, dis, iter):
    posb = pos.astype(jnp.bfloat16)
    bits = lax.bitcast_convert_type(posb, jnp.uint16).astype(jnp.uint32)
    packed = ((bits[:, 0] << 16) | bits[:, 1]).astype(jnp.int32)
    lr = _SCHED[iter]
    lr16 = jnp.full((16,), lr, dtype=jnp.float32)
    partials = _stress_partials(packed, i, j, dis, lr16)
    return jnp.sum(partials)


# speedup vs baseline: 1.0055x; 1.0055x over previous
import functools

import jax
import jax.numpy as jnp
from jax import lax
from jax.experimental import pallas as pl
from jax.experimental.pallas import tpu as pltpu
from jax.experimental.pallas import tpu_sc as plsc

_N = 100000
_E = 6400000
_NC, _NS = 2, 16
_NW = _NC * _NS
_EPW = _E // _NW
_CHUNK = 4000
_NCHUNK = _EPW // _CHUNK
_VPC = _CHUNK // 16

_SCHED = jnp.array([0.1], dtype=jnp.float32)

_MESH = plsc.VectorSubcoreMesh(core_axis_name="c", subcore_axis_name="s")


@functools.partial(
    pl.kernel,
    out_type=jax.ShapeDtypeStruct((_NW, 16), jnp.float32),
    mesh=_MESH,
    compiler_params=pltpu.CompilerParams(needs_layout_passes=False),
    scratch_types=[
        pltpu.VMEM((_N,), jnp.int32),
        pltpu.VMEM((_CHUNK,), jnp.int32),
        pltpu.VMEM((_CHUNK,), jnp.int32),
        pltpu.VMEM((_CHUNK,), jnp.int32),
        pltpu.VMEM((_CHUNK,), jnp.int32),
        pltpu.VMEM((_CHUNK,), jnp.float32),
        pltpu.VMEM((_CHUNK,), jnp.float32),
        pltpu.VMEM((16,), jnp.float32),
        pltpu.VMEM((16,), jnp.float32),
        pltpu.SemaphoreType.DMA,
        pltpu.SemaphoreType.DMA,
    ],
)
def _stress_partials(packed_hbm, i_hbm, j_hbm, dis_hbm, lr_hbm, out_hbm,
                     table_v, i0_v, i1_v, j0_v, j1_v, d0_v, d1_v,
                     lr_v, acc_v, sem0, sem1):
    cid = lax.axis_index("c")
    sid = lax.axis_index("s")
    wid = sid * _NC + cid
    base = wid * _EPW
    sems = [sem0, sem1]
    bufs = [(i0_v, j0_v, d0_v), (i1_v, j1_v, d1_v)]

    def _chunk_srcs(c):
        off = pl.multiple_of(base + c * _CHUNK, 8)
        sl = pl.ds(off, _CHUNK)
        return (i_hbm.at[sl], j_hbm.at[sl], dis_hbm.at[sl])

    def _start(c, b):
        for src, dst in zip(_chunk_srcs(c), bufs[b]):
            pltpu.async_copy(src, dst, sems[b])

    def _wait(c, b):
        for src, dst in zip(_chunk_srcs(c), bufs[b]):
            pltpu.make_async_copy(src, dst, sems[b]).wait()

    _start(0, 0)
    _start(1, 1)
    pltpu.sync_copy(packed_hbm, table_v)
    pltpu.sync_copy(lr_hbm, lr_v)
    lr = lr_v[...]
    acc_v[...] = jnp.zeros((16,), jnp.float32)

    half = jnp.float32(1.5)
    hmag = jnp.int32(0x5F3759DF)
    rmag = jnp.int32(0x7EF311C3)
    himask = jnp.int32(-65536)

    @pl.loop(0, _NCHUNK, step=2)
    def chunk_loop(c0):
        for b in range(2):
            c = c0 + b
            ib_v, jb_v, db_v = bufs[b]
            _wait(c, b)

            @plsc.parallel_loop(0, _VPC, unroll=8)
            def vec_loop(v):
                sl = pl.ds(v * 16, 16)
                iv = ib_v[sl]
                jv = jb_v[sl]
                dv = db_v[sl]
                wi = plsc.load_gather(table_v, [iv])
                wj = plsc.load_gather(table_v, [jv])
                xi = plsc.bitcast(wi & himask, jnp.float32)
                yi = plsc.bitcast(wi << 16, jnp.float32)
                xj = plsc.bitcast(wj & himask, jnp.float32)
                yj = plsc.bitcast(wj << 16, jnp.float32)
                dx = xi - xj
                dy = yi - yj
                s = dx * dx + jnp.float32(1e-18) + dy * dy
                r = plsc.bitcast(hmag - (plsc.bitcast(s, jnp.int32) >> 1),
                                 jnp.float32)
                r = r * (half - jnp.float32(0.5) * s * r * r)
                r = r * (half - jnp.float32(0.5) * s * r * r)
                mag = s * r
                m = jnp.maximum(dv, lr)
                q = plsc.bitcast(rmag - plsc.bitcast(m, jnp.int32), jnp.float32)
                q = q * (jnp.float32(2.0) - m * q)
                d = mag - dv
                plsc.addupdate(acc_v.at[:], (jnp.float32(0.25) * q) * (d * d))

            @pl.when(c + 2 < _NCHUNK)
            def _prefetch():
                _start(c + 2, b)

    pltpu.sync_copy(acc_v, out_hbm.at[wid])


def kernel(pos, i, j, vis_p_i, vis_p_j, dis, iter):
    posb = pos.astype(jnp.bfloat16)
    bits = lax.bitcast_convert_type(posb, jnp.uint16).astype(jnp.uint32)
    packed = ((bits[:, 0] << 16) | bits[:, 1]).astype(jnp.int32)
    lr = _SCHED[iter]
    lr16 = jnp.full((16,), lr, dtype=jnp.float32)
    partials = _stress_partials(packed, i, j, dis, lr16)
    return jnp.sum(partials)


# speedup vs baseline: 1.0443x; 1.0385x over previous
import functools

import jax
import jax.numpy as jnp
from jax import lax
from jax.experimental import pallas as pl
from jax.experimental.pallas import tpu as pltpu
from jax.experimental.pallas import tpu_sc as plsc

_N = 100000
_E = 6400000
_NC, _NS = 2, 16
_NW = _NC * _NS
_EPW = _E // _NW
_CHUNK = 4000
_NCHUNK = _EPW // _CHUNK
_VPC = _CHUNK // 16

_SCHED = jnp.array([0.1], dtype=jnp.float32)

_MESH = plsc.VectorSubcoreMesh(core_axis_name="c", subcore_axis_name="s")


@functools.partial(
    pl.kernel,
    out_type=jax.ShapeDtypeStruct((_NW, 16), jnp.float32),
    mesh=_MESH,
    compiler_params=pltpu.CompilerParams(needs_layout_passes=False),
    scratch_types=[
        pltpu.VMEM((_N,), jnp.int32),
        pltpu.VMEM((_CHUNK,), jnp.int32),
        pltpu.VMEM((_CHUNK,), jnp.int32),
        pltpu.VMEM((_CHUNK,), jnp.int32),
        pltpu.VMEM((_CHUNK,), jnp.int32),
        pltpu.VMEM((_CHUNK,), jnp.float32),
        pltpu.VMEM((_CHUNK,), jnp.float32),
        pltpu.VMEM((16,), jnp.float32),
        pltpu.VMEM((16,), jnp.float32),
        pltpu.SemaphoreType.DMA,
        pltpu.SemaphoreType.DMA,
    ],
)
def _stress_partials(packed_hbm, i_hbm, j_hbm, dis_hbm, lr_hbm, out_hbm,
                     table_v, i0_v, i1_v, j0_v, j1_v, d0_v, d1_v,
                     lr_v, acc_v, sem0, sem1):
    cid = lax.axis_index("c")
    sid = lax.axis_index("s")
    wid = sid * _NC + cid
    base = wid * _EPW
    sems = [sem0, sem1]
    bufs = [(i0_v, j0_v, d0_v), (i1_v, j1_v, d1_v)]

    def _chunk_srcs(c):
        off = pl.multiple_of(base + c * _CHUNK, 8)
        sl = pl.ds(off, _CHUNK)
        return (i_hbm.at[sl], j_hbm.at[sl], dis_hbm.at[sl])

    def _start(c, b):
        for src, dst in zip(_chunk_srcs(c), bufs[b]):
            pltpu.async_copy(src, dst, sems[b])

    def _wait(c, b):
        for src, dst in zip(_chunk_srcs(c), bufs[b]):
            pltpu.make_async_copy(src, dst, sems[b]).wait()

    _start(0, 0)
    _start(1, 1)
    pltpu.sync_copy(packed_hbm, table_v)
    pltpu.sync_copy(lr_hbm, lr_v)
    lr = lr_v[...]
    acc_v[...] = jnp.zeros((16,), jnp.float32)

    half = jnp.float32(1.5)
    hmag = jnp.int32(0x5F3759DF)
    rmag = jnp.int32(0x7EF311C3)
    himask = jnp.int32(-65536)

    @pl.loop(0, _NCHUNK, step=2)
    def chunk_loop(c0):
        for b in range(2):
            c = c0 + b
            ib_v, jb_v, db_v = bufs[b]
            _wait(c, b)

            @plsc.parallel_loop(0, _VPC, unroll=2)
            def vec_loop(v):
                sl = pl.ds(v * 16, 16)
                iv = ib_v[sl]
                jv = jb_v[sl]
                dv = db_v[sl]
                wi = plsc.load_gather(table_v, [iv])
                wj = plsc.load_gather(table_v, [jv])
                xi = plsc.bitcast(wi & himask, jnp.float32)
                yi = plsc.bitcast(wi << 16, jnp.float32)
                xj = plsc.bitcast(wj & himask, jnp.float32)
                yj = plsc.bitcast(wj << 16, jnp.float32)
                dx = xi - xj
                dy = yi - yj
                s = dx * dx + jnp.float32(1e-18) + dy * dy
                r = plsc.bitcast(hmag - (plsc.bitcast(s, jnp.int32) >> 1),
                                 jnp.float32)
                r = r * (half - jnp.float32(0.5) * s * r * r)
                r = r * (half - jnp.float32(0.5) * s * r * r)
                mag = s * r
                m = jnp.maximum(dv, lr)
                q = plsc.bitcast(rmag - plsc.bitcast(m, jnp.int32), jnp.float32)
                q = q * (jnp.float32(2.0) - m * q)
                d = mag - dv
                plsc.addupdate(acc_v.at[:], (jnp.float32(0.25) * q) * (d * d))

            @pl.when(c + 2 < _NCHUNK)
            def _prefetch():
                _start(c + 2, b)

    pltpu.sync_copy(acc_v, out_hbm.at[wid])


def kernel(pos, i, j, vis_p_i, vis_p_j, dis, iter):
    posb = pos.astype(jnp.bfloat16)
    bits = lax.bitcast_convert_type(posb, jnp.uint16).astype(jnp.uint32)
    packed = ((bits[:, 0] << 16) | bits[:, 1]).astype(jnp.int32)
    lr = _SCHED[iter]
    lr16 = jnp.full((16,), lr, dtype=jnp.float32)
    partials = _stress_partials(packed, i, j, dis, lr16)
    return jnp.sum(partials)


# speedup vs baseline: 1.0961x; 1.0496x over previous
import functools

import jax
import jax.numpy as jnp
from jax import lax
from jax.experimental import pallas as pl
from jax.experimental.pallas import tpu as pltpu
from jax.experimental.pallas import tpu_sc as plsc

_N = 100000
_E = 6400000
_NC, _NS = 2, 16
_NW = _NC * _NS
_EPW = _E // _NW
_CHUNK = 4000
_NCHUNK = _EPW // _CHUNK
_VPC = _CHUNK // 16

_SCHED = jnp.array([0.1], dtype=jnp.float32)

_MESH = plsc.VectorSubcoreMesh(core_axis_name="c", subcore_axis_name="s")


@functools.partial(
    pl.kernel,
    out_type=jax.ShapeDtypeStruct((_NW, 16), jnp.float32),
    mesh=_MESH,
    compiler_params=pltpu.CompilerParams(needs_layout_passes=False),
    scratch_types=[
        pltpu.VMEM((_N,), jnp.int32),
        pltpu.VMEM((_CHUNK,), jnp.int32),
        pltpu.VMEM((_CHUNK,), jnp.int32),
        pltpu.VMEM((_CHUNK,), jnp.int32),
        pltpu.VMEM((_CHUNK,), jnp.int32),
        pltpu.VMEM((_CHUNK,), jnp.float32),
        pltpu.VMEM((_CHUNK,), jnp.float32),
        pltpu.VMEM((16,), jnp.float32),
        pltpu.VMEM((16,), jnp.float32),
        pltpu.SemaphoreType.DMA,
        pltpu.SemaphoreType.DMA,
    ],
)
def _stress_partials(packed_hbm, i_hbm, j_hbm, dis_hbm, lr_hbm, out_hbm,
                     table_v, i0_v, i1_v, j0_v, j1_v, d0_v, d1_v,
                     lr_v, acc_v, sem0, sem1):
    cid = lax.axis_index("c")
    sid = lax.axis_index("s")
    wid = sid * _NC + cid
    base = wid * _EPW
    sems = [sem0, sem1]
    bufs = [(i0_v, j0_v, d0_v), (i1_v, j1_v, d1_v)]

    def _chunk_srcs(c):
        off = pl.multiple_of(base + c * _CHUNK, 8)
        sl = pl.ds(off, _CHUNK)
        return (i_hbm.at[sl], j_hbm.at[sl], dis_hbm.at[sl])

    def _start(c, b):
        for src, dst in zip(_chunk_srcs(c), bufs[b]):
            pltpu.async_copy(src, dst, sems[b])

    def _wait(c, b):
        for src, dst in zip(_chunk_srcs(c), bufs[b]):
            pltpu.make_async_copy(src, dst, sems[b]).wait()

    _start(0, 0)
    _start(1, 1)
    pltpu.sync_copy(packed_hbm, table_v)
    pltpu.sync_copy(lr_hbm, lr_v)
    lr = lr_v[...]
    acc_v[...] = jnp.zeros((16,), jnp.float32)

    half = jnp.float32(1.5)
    hmag = jnp.int32(0x5F3759DF)
    rmag = jnp.int32(0x7EF311C3)
    himask = jnp.int32(-65536)

    @pl.loop(0, _NCHUNK, step=2)
    def chunk_loop(c0):
        for b in range(2):
            c = c0 + b
            ib_v, jb_v, db_v = bufs[b]
            _wait(c, b)

            @plsc.parallel_loop(0, _VPC, unroll=4)
            def vec_loop(v):
                sl = pl.ds(v * 16, 16)
                iv = ib_v[sl]
                jv = jb_v[sl]
                dv = db_v[sl]
                wi = plsc.load_gather(table_v, [iv])
                wj = plsc.load_gather(table_v, [jv])
                bd = plsc.bitcast(plsc.bitcast(wi, jnp.bfloat16)
                                  - plsc.bitcast(wj, jnp.bfloat16), jnp.int32)
                dx = plsc.bitcast(bd & himask, jnp.float32)
                dy = plsc.bitcast(bd << 16, jnp.float32)
                s = dx * dx + jnp.float32(1e-18) + dy * dy
                r = plsc.bitcast(hmag - (plsc.bitcast(s, jnp.int32) >> 1),
                                 jnp.float32)
                r = r * (half - jnp.float32(0.5) * s * r * r)
                r = r * (half - jnp.float32(0.5) * s * r * r)
                mag = s * r
                m = jnp.maximum(dv, lr)
                q = plsc.bitcast(rmag - plsc.bitcast(m, jnp.int32), jnp.float32)
                q = q * (jnp.float32(2.0) - m * q)
                d = mag - dv
                plsc.addupdate(acc_v.at[:], (jnp.float32(0.25) * q) * (d * d))

            @pl.when(c + 2 < _NCHUNK)
            def _prefetch():
                _start(c + 2, b)

    pltpu.sync_copy(acc_v, out_hbm.at[wid])


def kernel(pos, i, j, vis_p_i, vis_p_j, dis, iter):
    posb = pos.astype(jnp.bfloat16)
    bits = lax.bitcast_convert_type(posb, jnp.uint16).astype(jnp.uint32)
    packed = ((bits[:, 0] << 16) | bits[:, 1]).astype(jnp.int32)
    lr = _SCHED[iter]
    lr16 = jnp.full((16,), lr, dtype=jnp.float32)
    partials = _stress_partials(packed, i, j, dis, lr16)
    return jnp.sum(partials)


# speedup vs baseline: 1.2314x; 1.1235x over previous
import functools

import jax
import jax.numpy as jnp
from jax import lax
from jax.experimental import pallas as pl
from jax.experimental.pallas import tpu as pltpu
from jax.experimental.pallas import tpu_sc as plsc

_N = 100000
_E = 6400000
_NC, _NS = 2, 16
_NW = _NC * _NS
_EPW = _E // _NW
_CHUNK = 4000
_NCHUNK = _EPW // _CHUNK
_VPC = _CHUNK // 16

_SCHED = jnp.array([0.1], dtype=jnp.float32)

_MESH = plsc.VectorSubcoreMesh(core_axis_name="c", subcore_axis_name="s")


@functools.partial(
    pl.kernel,
    out_type=jax.ShapeDtypeStruct((_NW, 16), jnp.float32),
    mesh=_MESH,
    compiler_params=pltpu.CompilerParams(needs_layout_passes=False),
    scratch_types=[
        pltpu.VMEM((_N,), jnp.int32),
        pltpu.VMEM((_CHUNK,), jnp.int32),
        pltpu.VMEM((_CHUNK,), jnp.int32),
        pltpu.VMEM((_CHUNK,), jnp.int32),
        pltpu.VMEM((_CHUNK,), jnp.int32),
        pltpu.VMEM((_CHUNK,), jnp.float32),
        pltpu.VMEM((_CHUNK,), jnp.float32),
        pltpu.VMEM((16,), jnp.float32),
        pltpu.VMEM((16,), jnp.float32),
        pltpu.SemaphoreType.DMA,
        pltpu.SemaphoreType.DMA,
    ],
)
def _stress_partials(packed_hbm, i_hbm, j_hbm, dis_hbm, lr_hbm, out_hbm,
                     table_v, i0_v, i1_v, j0_v, j1_v, d0_v, d1_v,
                     lr_v, acc_v, sem0, sem1):
    cid = lax.axis_index("c")
    sid = lax.axis_index("s")
    wid = sid * _NC + cid
    base = wid * _EPW
    sems = [sem0, sem1]
    bufs = [(i0_v, j0_v, d0_v), (i1_v, j1_v, d1_v)]

    def _chunk_srcs(c):
        off = pl.multiple_of(base + c * _CHUNK, 8)
        sl = pl.ds(off, _CHUNK)
        return (i_hbm.at[sl], j_hbm.at[sl], dis_hbm.at[sl])

    def _start(c, b):
        for src, dst in zip(_chunk_srcs(c), bufs[b]):
            pltpu.async_copy(src, dst, sems[b])

    def _wait(c, b):
        for src, dst in zip(_chunk_srcs(c), bufs[b]):
            pltpu.make_async_copy(src, dst, sems[b]).wait()

    _start(0, 0)
    _start(1, 1)
    pltpu.sync_copy(packed_hbm, table_v)
    pltpu.sync_copy(lr_hbm, lr_v)
    lr = lr_v[...]
    acc_v[...] = jnp.zeros((16,), jnp.float32)

    hmag = jnp.int32(0x5F1FFFF9)
    rmag = jnp.int32(0x7EF311C3)
    himask = jnp.int32(-65536)

    @pl.loop(0, _NCHUNK, step=2)
    def chunk_loop(c0):
        for b in range(2):
            c = c0 + b
            ib_v, jb_v, db_v = bufs[b]
            _wait(c, b)

            @plsc.parallel_loop(0, _VPC, unroll=4)
            def vec_loop(v):
                sl = pl.ds(v * 16, 16)
                iv = ib_v[sl]
                jv = jb_v[sl]
                dv = db_v[sl]
                wi = plsc.load_gather(table_v, [iv])
                wj = plsc.load_gather(table_v, [jv])
                bd = plsc.bitcast(plsc.bitcast(wi, jnp.bfloat16)
                                  - plsc.bitcast(wj, jnp.bfloat16), jnp.int32)
                dx = plsc.bitcast(bd & himask, jnp.float32)
                dy = plsc.bitcast(bd << 16, jnp.float32)
                s = dx * dx + jnp.float32(1e-18) + dy * dy
                r = plsc.bitcast(hmag - (plsc.bitcast(s, jnp.int32) >> 1),
                                 jnp.float32)
                u = s * r
                w = jnp.float32(2.38924456) - u * r
                mag = jnp.float32(0.703952253) * (u * w)
                m = jnp.maximum(dv, lr)
                q = plsc.bitcast(rmag - plsc.bitcast(m, jnp.int32), jnp.float32)
                q = q * (jnp.float32(2.0) - m * q)
                d = mag - dv
                plsc.addupdate(acc_v.at[:], (jnp.float32(0.25) * q) * (d * d))

            @pl.when(c + 2 < _NCHUNK)
            def _prefetch():
                _start(c + 2, b)

    pltpu.sync_copy(acc_v, out_hbm.at[wid])


def kernel(pos, i, j, vis_p_i, vis_p_j, dis, iter):
    posb = pos.astype(jnp.bfloat16)
    bits = lax.bitcast_convert_type(posb, jnp.uint16).astype(jnp.uint32)
    packed = ((bits[:, 0] << 16) | bits[:, 1]).astype(jnp.int32)
    lr = _SCHED[iter]
    lr16 = jnp.full((16,), lr, dtype=jnp.float32)
    partials = _stress_partials(packed, i, j, dis, lr16)
    return jnp.sum(partials)
